# Initial kernel scaffold; baseline (speedup 1.0000x reference)
import jax
import jax.numpy as jnp
from jax.experimental import pallas as pl


def kernel(edge_index, row_embed, col_embed, W0, b0, W1, b1, W2, b2):
    raise NotImplementedError("write your pallas kernel here")


# speedup vs baseline: 12.1077x; 12.1077x over previous
import functools

import jax
import jax.numpy as jnp
from jax import lax
from jax.experimental import pallas as pl
from jax.experimental.pallas import tpu as pltpu
from jax.experimental.pallas import tpu_sc as plsc

D = 32
CH = 128


@functools.lru_cache(maxsize=None)
def _make_propagate(n_nodes, e_pad):
    info = plsc.get_sparse_core_info()
    nc, ns = info.num_cores, info.num_subcores
    nw = nc * ns
    per_w = e_pad // nw
    n_iter = per_w // CH
    assert per_w % CH == 0 and e_pad % nw == 0
    agg_rows = per_w
    assert agg_rows >= n_nodes + 1 and agg_rows % ns == 0
    zrows = agg_rows // ns
    zb = zrows // 8
    assert zrows % zb == 0
    rows_out = -(-(-(-n_nodes // ns)) // 8) * 8
    n_out = rows_out * ns
    assert n_out >= n_nodes and agg_rows >= n_out

    mesh = plsc.VectorSubcoreMesh(core_axis_name="c", subcore_axis_name="s")

    @functools.partial(
        pl.kernel,
        mesh=mesh,
        out_type=jax.ShapeDtypeStruct((nc, n_out, D), jnp.float32),
        compiler_params=pltpu.CompilerParams(use_tc_tiling_on_sc=False),
        scratch_types=[
            pltpu.VMEM((CH,), jnp.int32),
            pltpu.VMEM((CH,), jnp.int32),
            pltpu.VMEM((CH, D), jnp.float32),
            pltpu.VMEM((zb, D), jnp.float32),
            pltpu.VMEM_SHARED((agg_rows, D), jnp.float32),
            pltpu.SemaphoreType.DMA,
        ],
    )
    def propagate(src_hbm, dst_hbm, h_hbm, out_hbm,
                  src_v, dst_v, msgs_v, zero_v, agg_sh, sem):
        c = lax.axis_index("c")
        s = lax.axis_index("s")
        wid = s * nc + c

        zvec = jnp.zeros((16,), jnp.float32)

        def zrow(r, carry):
            zero_v[r, pl.ds(0, 16)] = zvec
            zero_v[r, pl.ds(16, 16)] = zvec
            return carry

        lax.fori_loop(0, zb, zrow, 0)

        def zcopy(i, carry):
            pltpu.sync_copy(zero_v, agg_sh.at[pl.ds(s * zrows + i * zb, zb)])
            return carry

        lax.fori_loop(0, zrows // zb, zcopy, 0)
        plsc.subcore_barrier()

        base = wid * per_w

        def edge_step(i, carry):
            off = base + i * CH
            pltpu.sync_copy(src_hbm.at[pl.ds(off, CH)], src_v)
            pltpu.sync_copy(dst_hbm.at[pl.ds(off, CH)], dst_v)
            pltpu.async_copy(h_hbm.at[src_v], msgs_v, sem).wait()
            pltpu.sync_copy(msgs_v, agg_sh.at[dst_v], add=True)
            return carry

        lax.fori_loop(0, n_iter, edge_step, 0)
        plsc.subcore_barrier()

        r0 = s * rows_out
        pltpu.sync_copy(agg_sh.at[pl.ds(r0, rows_out)],
                        out_hbm.at[c, pl.ds(r0, rows_out)])

    return propagate


_DN = (((1,), (1,)), ((), ()))


def _lin_body(x_ref, w_ref, b_ref, o_ref):
    o_ref[...] = lax.dot_general(
        x_ref[...], w_ref[...], _DN, preferred_element_type=jnp.float32
    ) + b_ref[...]


def _fuse_body(p_ref, h_ref, w_ref, b_ref, o_ref):
    x = jax.nn.relu(p_ref[0] + p_ref[1] + h_ref[...])
    o_ref[...] = lax.dot_general(
        x, w_ref[...], _DN, preferred_element_type=jnp.float32
    ) + b_ref[...]


def _final_body(p_ref, h_ref, o_ref):
    o_ref[...] = jax.nn.relu(p_ref[0] + p_ref[1] + h_ref[...])


def _row_grid(n_rows, blk):
    assert n_rows % blk == 0
    return n_rows // blk


_BLK = 2000


def _lin(x, w, b):
    n = x.shape[0]
    return pl.pallas_call(
        _lin_body,
        grid=(_row_grid(n, _BLK),),
        in_specs=[
            pl.BlockSpec((_BLK, D), lambda i: (i, 0)),
            pl.BlockSpec((D, D), lambda i: (0, 0)),
            pl.BlockSpec((1, D), lambda i: (0, 0)),
        ],
        out_specs=pl.BlockSpec((_BLK, D), lambda i: (i, 0)),
        out_shape=jax.ShapeDtypeStruct((n, D), jnp.float32),
    )(x, w, b.reshape(1, D))


def _fuse(p, h, w, b):
    n = h.shape[0]
    return pl.pallas_call(
        _fuse_body,
        grid=(_row_grid(n, _BLK),),
        in_specs=[
            pl.BlockSpec((2, _BLK, D), lambda i: (0, i, 0)),
            pl.BlockSpec((_BLK, D), lambda i: (i, 0)),
            pl.BlockSpec((D, D), lambda i: (0, 0)),
            pl.BlockSpec((1, D), lambda i: (0, 0)),
        ],
        out_specs=pl.BlockSpec((_BLK, D), lambda i: (i, 0)),
        out_shape=jax.ShapeDtypeStruct((n, D), jnp.float32),
    )(p, h, w, b.reshape(1, D))


def _final(p, h):
    n = h.shape[0]
    return pl.pallas_call(
        _final_body,
        grid=(_row_grid(n, _BLK),),
        in_specs=[
            pl.BlockSpec((2, _BLK, D), lambda i: (0, i, 0)),
            pl.BlockSpec((_BLK, D), lambda i: (i, 0)),
        ],
        out_specs=pl.BlockSpec((_BLK, D), lambda i: (i, 0)),
        out_shape=jax.ShapeDtypeStruct((n, D), jnp.float32),
    )(p, h)


def kernel(edge_index, row_embed, col_embed, W0, b0, W1, b1, W2, b2):
    n_rows = row_embed.shape[0]
    n_cols = col_embed.shape[0]
    n_nodes = n_rows + n_cols
    e = edge_index.shape[1]

    nw = 32
    per_w = -(-e // (nw * CH)) * CH
    e_pad = per_w * nw
    pad = e_pad - e

    src = edge_index[0].astype(jnp.int32)
    dst = edge_index[1].astype(jnp.int32)
    if pad:
        src = jnp.concatenate([src, jnp.zeros((pad,), jnp.int32)])
        dst = jnp.concatenate([dst, jnp.full((pad,), n_nodes, jnp.int32)])

    propagate = _make_propagate(n_nodes, e_pad)

    x0 = jnp.concatenate([row_embed, col_embed], axis=0)
    h = _lin(x0, W0, b0)
    p = propagate(src, dst, h)
    h = _fuse(p, h, W1, b1)
    p = propagate(src, dst, h)
    h = _fuse(p, h, W2, b2)
    p = propagate(src, dst, h)
    x3 = _final(p, h)
    return (x3[:n_rows], x3[n_rows:])
